# Initial kernel scaffold; baseline (speedup 1.0000x reference)
import jax
import jax.numpy as jnp
from jax.experimental import pallas as pl


def kernel(x, spatial_w, spatial_b, conv_in_w, conv_w, conv_b, conv_out_w):
    raise NotImplementedError("write your pallas kernel here")


# speedup vs baseline: 1.0061x; 1.0061x over previous
import functools
import math

import jax
import jax.numpy as jnp
from jax import lax
from jax.experimental import pallas as pl
from jax.experimental.pallas import tpu as pltpu


def _params(num_axes, vmem_mb):
    return pltpu.CompilerParams(
        dimension_semantics=("parallel",) * num_axes,
        vmem_limit_bytes=vmem_mb << 20,
    )


def _conv_in_body(xp_ref, w9_ref, b_ref, win_ref, osp_ref, oxf_ref, *,
                  wp, lc):
    ctr = None
    acc = b_ref[...].astype(jnp.float32)
    for k in range(9):
        ky, kx = divmod(k, 3)
        tap = xp_ref[0, :, ky * wp + kx:ky * wp + kx + lc]
        if k == 4:
            ctr = tap
        acc = acc + jnp.dot(w9_ref[k], tap,
                            preferred_element_type=jnp.float32)
    osp_ref[0] = acc.astype(osp_ref.dtype)
    xf = jnp.dot(win_ref[...], ctr, preferred_element_type=jnp.float32)
    oxf_ref[0] = jnp.maximum(xf, 0.0)


def _conv_in(x_pad_flat, w9, b_sp, w_in, *, wp, lc):
    n, c_in, lpad = x_pad_flat.shape
    c_out = w9.shape[1]
    mid = w_in.shape[0]
    return pl.pallas_call(
        functools.partial(_conv_in_body, wp=wp, lc=lc),
        out_shape=(jax.ShapeDtypeStruct((n, c_out, lc), jnp.bfloat16),
                   jax.ShapeDtypeStruct((n, mid, lc), jnp.float32)),
        grid=(n,),
        in_specs=[
            pl.BlockSpec((1, c_in, lpad), lambda i: (i, 0, 0)),
            pl.BlockSpec((9, c_out, c_in), lambda i: (0, 0, 0)),
            pl.BlockSpec((c_out, 1), lambda i: (0, 0)),
            pl.BlockSpec((mid, c_in), lambda i: (0, 0)),
        ],
        out_specs=[
            pl.BlockSpec((1, c_out, lc), lambda i: (i, 0, 0)),
            pl.BlockSpec((1, mid, lc), lambda i: (i, 0, 0)),
        ],
        compiler_params=_params(1, 48),
    )(x_pad_flat, w9, b_sp, w_in)


def _freq_body(s_ref, wc_ref, b_ref, o_ref):
    s = s_ref[0].astype(jnp.bfloat16)
    z = jnp.dot(wc_ref[...], s, preferred_element_type=jnp.float32)
    o_ref[0] = jnp.maximum(z + b_ref[...].astype(jnp.float32), 0.0)


def _freq_conv(stacked, wc2, b2):
    n, c2, m = stacked.shape
    return pl.pallas_call(
        _freq_body,
        out_shape=jax.ShapeDtypeStruct((n, c2, m), jnp.float32),
        grid=(n,),
        in_specs=[
            pl.BlockSpec((1, c2, m), lambda i: (i, 0, 0)),
            pl.BlockSpec((c2, c2), lambda i: (0, 0)),
            pl.BlockSpec((c2, 1), lambda i: (0, 0)),
        ],
        out_specs=pl.BlockSpec((1, c2, m), lambda i: (i, 0, 0)),
        compiler_params=_params(1, 32),
    )(stacked, wc2, b2)


def _final_body(xf_ref, of_ref, sp_ref, w_ref, o_ref):
    s = (xf_ref[0] + of_ref[0]).astype(jnp.bfloat16)
    acc = jnp.dot(w_ref[...], s, preferred_element_type=jnp.float32)
    o_ref[0] = acc + sp_ref[0].astype(jnp.float32)


def _final_conv(x_fft, out_fft, out_sp, w_out):
    n, mid, m = x_fft.shape
    c_out = w_out.shape[0]
    return pl.pallas_call(
        _final_body,
        out_shape=jax.ShapeDtypeStruct((n, c_out, m), jnp.float32),
        grid=(n,),
        in_specs=[
            pl.BlockSpec((1, mid, m), lambda i: (i, 0, 0)),
            pl.BlockSpec((1, mid, m), lambda i: (i, 0, 0)),
            pl.BlockSpec((1, c_out, m), lambda i: (i, 0, 0)),
            pl.BlockSpec((c_out, mid), lambda i: (0, 0)),
        ],
        out_specs=pl.BlockSpec((1, c_out, m), lambda i: (i, 0, 0)),
        compiler_params=_params(1, 32),
    )(x_fft, out_fft, out_sp, w_out)


def kernel(x, spatial_w, spatial_b, conv_in_w, conv_w, conv_b, conv_out_w):
    n, c_in, h, w = x.shape
    c_out = spatial_w.shape[0]
    mid = conv_in_w.shape[0]
    hp, wp = h + 2, w + 2
    hw = h * w
    lc = h * wp
    lpad = -(-(2 * wp + 2 + lc) // 128) * 128

    x_pad = jnp.pad(x, ((0, 0), (0, 0), (1, 1), (1, 1)), mode="reflect")
    x_pad_flat = x_pad.reshape(n, c_in, hp * wp).astype(jnp.bfloat16)
    x_pad_flat = jnp.pad(x_pad_flat, ((0, 0), (0, 0), (0, lpad - hp * wp)))

    w9 = (jnp.transpose(spatial_w, (2, 3, 0, 1))
          .reshape(9, c_out, c_in).astype(jnp.bfloat16))
    w_in = conv_in_w.astype(jnp.bfloat16)

    osp_pf, oxf_pf = _conv_in(
        x_pad_flat, w9, spatial_b.reshape(c_out, 1), w_in, wp=wp, lc=lc)

    osp = osp_pf.reshape(n, c_out, h, wp)[:, :, :, :w]
    xf = oxf_pf.reshape(n, mid, h, wp)[:, :, :, :w]

    ffted = jnp.fft.rfftn(xf, axes=(-2, -1), norm="ortho")
    wf = ffted.shape[-1]
    m = h * wf
    re = jnp.real(ffted).reshape(n, mid, m)
    im = jnp.imag(ffted).reshape(n, mid, m)
    stacked = jnp.concatenate([re, im], axis=1)

    w_rr, w_ri = conv_w[0::2, 0::2], conv_w[0::2, 1::2]
    w_ir, w_ii = conv_w[1::2, 0::2], conv_w[1::2, 1::2]
    wc2 = jnp.concatenate(
        [jnp.concatenate([w_rr, w_ri], axis=1),
         jnp.concatenate([w_ir, w_ii], axis=1)], axis=0).astype(jnp.bfloat16)
    b2 = jnp.concatenate([conv_b[0::2], conv_b[1::2]]).reshape(2 * mid, 1)

    zo = _freq_conv(stacked, wc2, b2)
    cplx = lax.complex(zo[:, :mid].reshape(n, mid, h, wf),
                       zo[:, mid:].reshape(n, mid, h, wf))
    out_fft = jnp.fft.irfftn(cplx, s=(h, w), axes=(-2, -1), norm="ortho")

    out = _final_conv(xf.reshape(n, mid, hw),
                      out_fft.reshape(n, mid, hw),
                      osp.reshape(n, c_out, hw),
                      conv_out_w.astype(jnp.bfloat16))
    return out.reshape(n, c_out, h, w)


# speedup vs baseline: 1.6310x; 1.6211x over previous
import functools
import math

import jax
import jax.numpy as jnp
from jax.experimental import pallas as pl
from jax.experimental.pallas import tpu as pltpu


def _params(num_axes, vmem_mb):
    return pltpu.CompilerParams(
        dimension_semantics=("parallel",) * num_axes,
        vmem_limit_bytes=vmem_mb << 20,
    )


def _conv_in_body(xp_ref, w9_ref, b_ref, win_ref, osp_ref, oxf_ref, *,
                  wp, lc):
    ctr = None
    acc = b_ref[...].astype(jnp.float32)
    for k in range(9):
        ky, kx = divmod(k, 3)
        tap = xp_ref[0, :, ky * wp + kx:ky * wp + kx + lc]
        if k == 4:
            ctr = tap
        acc = acc + jnp.dot(w9_ref[k], tap,
                            preferred_element_type=jnp.float32)
    osp_ref[0] = acc.astype(osp_ref.dtype)
    xf = jnp.dot(win_ref[...], ctr, preferred_element_type=jnp.float32)
    oxf_ref[0] = jnp.maximum(xf, 0.0).astype(oxf_ref.dtype)


def _conv_in(x_pad_flat, w9, b_sp, w_in, *, wp, lc):
    n, c_in, lpad = x_pad_flat.shape
    c_out = w9.shape[1]
    mid = w_in.shape[0]
    return pl.pallas_call(
        functools.partial(_conv_in_body, wp=wp, lc=lc),
        out_shape=(jax.ShapeDtypeStruct((n, c_out, lc), jnp.bfloat16),
                   jax.ShapeDtypeStruct((n, mid, lc), jnp.bfloat16)),
        grid=(n,),
        in_specs=[
            pl.BlockSpec((1, c_in, lpad), lambda i: (i, 0, 0)),
            pl.BlockSpec((9, c_out, c_in), lambda i: (0, 0, 0)),
            pl.BlockSpec((c_out, 1), lambda i: (0, 0)),
            pl.BlockSpec((mid, c_in), lambda i: (0, 0)),
        ],
        out_specs=[
            pl.BlockSpec((1, c_out, lc), lambda i: (i, 0, 0)),
            pl.BlockSpec((1, mid, lc), lambda i: (i, 0, 0)),
        ],
        compiler_params=_params(1, 48),
    )(x_pad_flat, w9, b_sp, w_in)


def _mm_body(x_ref, m_ref, o_ref):
    o_ref[0] = jnp.dot(x_ref[0], m_ref[...],
                       preferred_element_type=jnp.float32).astype(o_ref.dtype)


def _dft_fwd(xf, mf, *, jt):
    n, mid, hw = xf.shape
    cols = mf.shape[1]
    nj = cols // jt
    return pl.pallas_call(
        _mm_body,
        out_shape=jax.ShapeDtypeStruct((n, mid, cols), jnp.bfloat16),
        grid=(nj, n),
        in_specs=[
            pl.BlockSpec((1, mid, hw), lambda j, i: (i, 0, 0)),
            pl.BlockSpec((hw, jt), lambda j, i: (0, j)),
        ],
        out_specs=pl.BlockSpec((1, mid, jt), lambda j, i: (i, 0, j)),
        compiler_params=_params(2, 48),
    )(xf, mf)


def _freq_body(z_ref, wrr_ref, wri_ref, wir_ref, wii_ref, br_ref, bi_ref,
               o_ref, *, mh):
    zre = z_ref[0][:, :mh]
    zim = z_ref[0][:, mh:]
    ore = (jnp.dot(wrr_ref[...], zre, preferred_element_type=jnp.float32)
           + jnp.dot(wri_ref[...], zim, preferred_element_type=jnp.float32)
           + br_ref[...].astype(jnp.float32))
    oim = (jnp.dot(wir_ref[...], zre, preferred_element_type=jnp.float32)
           + jnp.dot(wii_ref[...], zim, preferred_element_type=jnp.float32)
           + bi_ref[...].astype(jnp.float32))
    o_ref[0, :, :mh] = jnp.maximum(ore, 0.0).astype(o_ref.dtype)
    o_ref[0, :, mh:] = jnp.maximum(oim, 0.0).astype(o_ref.dtype)


def _freq_conv(y, w4, b2, *, mh):
    n, mid, cols = y.shape
    wrr, wri, wir, wii = w4
    br, bi = b2
    w_spec = pl.BlockSpec((mid, mid), lambda i: (0, 0))
    b_spec = pl.BlockSpec((mid, 1), lambda i: (0, 0))
    return pl.pallas_call(
        functools.partial(_freq_body, mh=mh),
        out_shape=jax.ShapeDtypeStruct((n, mid, cols), jnp.bfloat16),
        grid=(n,),
        in_specs=[pl.BlockSpec((1, mid, cols), lambda i: (i, 0, 0)),
                  w_spec, w_spec, w_spec, w_spec, b_spec, b_spec],
        out_specs=pl.BlockSpec((1, mid, cols), lambda i: (i, 0, 0)),
        compiler_params=_params(1, 32),
    )(y, wrr, wri, wir, wii, br, bi)


def _inv_final_body(z_ref, mi_ref, xf_ref, sp_ref, w_ref, o_ref):
    offt = jnp.dot(z_ref[0], mi_ref[...],
                   preferred_element_type=jnp.float32)
    s = (xf_ref[0].astype(jnp.float32) + offt).astype(jnp.bfloat16)
    acc = jnp.dot(w_ref[...], s, preferred_element_type=jnp.float32)
    o_ref[0] = acc + sp_ref[0].astype(jnp.float32)


def _inv_final(z, mi, xf, osp, w_out, *, jt):
    n, mid, cols = z.shape
    hw = mi.shape[1]
    c_out = w_out.shape[0]
    nj = hw // jt
    return pl.pallas_call(
        _inv_final_body,
        out_shape=jax.ShapeDtypeStruct((n, c_out, hw), jnp.float32),
        grid=(nj, n),
        in_specs=[
            pl.BlockSpec((1, mid, cols), lambda j, i: (i, 0, 0)),
            pl.BlockSpec((cols, jt), lambda j, i: (0, j)),
            pl.BlockSpec((1, mid, jt), lambda j, i: (i, 0, j)),
            pl.BlockSpec((1, c_out, jt), lambda j, i: (i, 0, j)),
            pl.BlockSpec((c_out, mid), lambda j, i: (0, 0)),
        ],
        out_specs=pl.BlockSpec((1, c_out, jt), lambda j, i: (i, 0, j)),
        compiler_params=_params(2, 48),
    )(z, mi, xf, osp, w_out)


def kernel(x, spatial_w, spatial_b, conv_in_w, conv_w, conv_b, conv_out_w):
    n, c_in, h, w = x.shape
    c_out = spatial_w.shape[0]
    mid = conv_in_w.shape[0]
    hp, wp = h + 2, w + 2
    hw = h * w
    wf = w // 2 + 1
    fpad = wf + (-(h * wf) % 128) // h
    mh = h * fpad
    lc = h * wp
    lpad = -(-(2 * wp + 2 + lc) // 128) * 128

    f32 = jnp.float32
    x_pad = jnp.pad(x, ((0, 0), (0, 0), (1, 1), (1, 1)), mode="reflect")
    x_pad_flat = x_pad.reshape(n, c_in, hp * wp).astype(jnp.bfloat16)
    x_pad_flat = jnp.pad(x_pad_flat, ((0, 0), (0, 0), (0, lpad - hp * wp)))

    w9 = (jnp.transpose(spatial_w, (2, 3, 0, 1))
          .reshape(9, c_out, c_in).astype(jnp.bfloat16))

    osp_pf, oxf_pf = _conv_in(
        x_pad_flat, w9, spatial_b.reshape(c_out, 1),
        conv_in_w.astype(jnp.bfloat16), wp=wp, lc=lc)

    osp = osp_pf.reshape(n, c_out, h, wp)[:, :, :, :w].reshape(n, c_out, hw)
    xf = oxf_pf.reshape(n, mid, h, wp)[:, :, :, :w].reshape(n, mid, hw)

    inv_s = 1.0 / math.sqrt(hw)
    ang_h = (2.0 * math.pi / h) * jnp.outer(jnp.arange(h), jnp.arange(h))
    ang_w = (2.0 * math.pi / w) * jnp.outer(jnp.arange(w), jnp.arange(wf))
    ch, sh = jnp.cos(ang_h), jnp.sin(ang_h)
    cw, sw = jnp.cos(ang_w), jnp.sin(ang_w)
    zpad = ((0, 0), (0, fpad - wf))
    cwp, swp = jnp.pad(cw, zpad), jnp.pad(sw, zpad)

    m_re = (jnp.einsum("yk,xf->ykxf", ch, cwp)
            - jnp.einsum("yk,xf->ykxf", sh, swp)).transpose(0, 2, 1, 3)
    m_im = -(jnp.einsum("yk,xf->ykxf", sh, cwp)
             + jnp.einsum("yk,xf->ykxf", ch, swp)).transpose(0, 2, 1, 3)
    mf = (inv_s * jnp.concatenate(
        [m_re.reshape(hw, mh), m_im.reshape(hw, mh)], axis=1)
    ).astype(jnp.bfloat16)

    y = _dft_fwd(xf, mf, jt=mh)

    w4 = tuple(m.astype(jnp.bfloat16)
               for m in (conv_w[0::2, 0::2], conv_w[0::2, 1::2],
                         conv_w[1::2, 0::2], conv_w[1::2, 1::2]))
    b2 = (conv_b[0::2].reshape(mid, 1), conv_b[1::2].reshape(mid, 1))
    z = _freq_conv(y, w4, b2, mh=mh)

    fi = jnp.arange(fpad)
    wt = jnp.where((fi == 0) | (fi == wf - 1), 1.0, 2.0)
    wt = jnp.where(fi < wf, wt, 0.0)
    cwi = cwp * wt
    swi = swp * wt
    mi_re = (jnp.einsum("yk,xf->kfyx", ch, cwi)
             - jnp.einsum("yk,xf->kfyx", sh, swi))
    mi_im = -(jnp.einsum("yk,xf->kfyx", sh, cwi)
              + jnp.einsum("yk,xf->kfyx", ch, swi))
    mi = (inv_s * jnp.concatenate(
        [mi_re.reshape(mh, hw), mi_im.reshape(mh, hw)], axis=0)
    ).astype(jnp.bfloat16)

    out = _inv_final(z, mi, xf, osp, conv_out_w.astype(jnp.bfloat16),
                     jt=hw // 2)
    return out.reshape(n, c_out, h, w)


# speedup vs baseline: 2.2529x; 1.3812x over previous
import functools
import math

import jax
import jax.numpy as jnp
import numpy as np
from jax.experimental import pallas as pl
from jax.experimental.pallas import tpu as pltpu


@functools.lru_cache(maxsize=4)
def _dft_mats(h, w):
    hw = h * w
    wf = w // 2 + 1
    fpad = wf + (-(h * wf) % 128) // h
    mh = h * fpad
    inv_s = 1.0 / math.sqrt(hw)

    ang_h = (2.0 * np.pi / h) * np.outer(np.arange(h), np.arange(h))
    ang_w = (2.0 * np.pi / w) * np.outer(np.arange(w), np.arange(wf))
    ch, sh = np.cos(ang_h), np.sin(ang_h)
    cw, sw = np.cos(ang_w), np.sin(ang_w)
    zpad = ((0, 0), (0, fpad - wf))
    cwp, swp = np.pad(cw, zpad), np.pad(sw, zpad)

    m_re = (np.einsum("yk,xf->yxkf", ch, cwp)
            - np.einsum("yk,xf->yxkf", sh, swp))
    m_im = -(np.einsum("yk,xf->yxkf", sh, cwp)
             + np.einsum("yk,xf->yxkf", ch, swp))
    mf = inv_s * np.concatenate(
        [m_re.reshape(hw, mh), m_im.reshape(hw, mh)], axis=1)

    fi = np.arange(fpad)
    wt = np.where((fi == 0) | (fi == wf - 1), 1.0, 2.0)
    wt = np.where(fi < wf, wt, 0.0)
    cwi, swi = cwp * wt, swp * wt
    mi_re = (np.einsum("yk,xf->kfyx", ch, cwi)
             - np.einsum("yk,xf->kfyx", sh, swi))
    mi_im = -(np.einsum("yk,xf->kfyx", sh, cwi)
              + np.einsum("yk,xf->kfyx", ch, swi))
    mi = inv_s * np.concatenate(
        [mi_re.reshape(mh, hw), mi_im.reshape(mh, hw)], axis=0)

    bf16 = jnp.bfloat16
    return (np.asarray(mf, np.float32).astype(bf16),
            np.asarray(mi, np.float32).astype(bf16), fpad, mh)


def _params(num_axes, vmem_mb):
    return pltpu.CompilerParams(
        dimension_semantics=("parallel",) * num_axes,
        vmem_limit_bytes=vmem_mb << 20,
    )


def _conv_in_body(xp_ref, w9_ref, b_ref, win_ref, osp_ref, oxf_ref, *,
                  wp, lc):
    ctr = None
    acc = b_ref[...].astype(jnp.float32)
    for k in range(9):
        ky, kx = divmod(k, 3)
        tap = xp_ref[0, :, ky * wp + kx:ky * wp + kx + lc]
        if k == 4:
            ctr = tap
        acc = acc + jnp.dot(w9_ref[k], tap,
                            preferred_element_type=jnp.float32)
    osp_ref[0] = acc.astype(osp_ref.dtype)
    xf = jnp.dot(win_ref[...], ctr, preferred_element_type=jnp.float32)
    oxf_ref[0] = jnp.maximum(xf, 0.0).astype(oxf_ref.dtype)


def _conv_in(x_pad_flat, w9, b_sp, w_in, *, wp, lc):
    n, c_in, lpad = x_pad_flat.shape
    c_out = w9.shape[1]
    mid = w_in.shape[0]
    return pl.pallas_call(
        functools.partial(_conv_in_body, wp=wp, lc=lc),
        out_shape=(jax.ShapeDtypeStruct((n, c_out, lc), jnp.bfloat16),
                   jax.ShapeDtypeStruct((n, mid, lc), jnp.bfloat16)),
        grid=(n,),
        in_specs=[
            pl.BlockSpec((1, c_in, lpad), lambda i: (i, 0, 0)),
            pl.BlockSpec((9, c_out, c_in), lambda i: (0, 0, 0)),
            pl.BlockSpec((c_out, 1), lambda i: (0, 0)),
            pl.BlockSpec((mid, c_in), lambda i: (0, 0)),
        ],
        out_specs=[
            pl.BlockSpec((1, c_out, lc), lambda i: (i, 0, 0)),
            pl.BlockSpec((1, mid, lc), lambda i: (i, 0, 0)),
        ],
        compiler_params=_params(1, 48),
    )(x_pad_flat, w9, b_sp, w_in)


def _mm_body(x_ref, m_ref, o_ref):
    o_ref[0] = jnp.dot(x_ref[0], m_ref[...],
                       preferred_element_type=jnp.float32).astype(o_ref.dtype)


def _dft_fwd(xf, mf, *, jt):
    n, mid, hw = xf.shape
    cols = mf.shape[1]
    nj = cols // jt
    return pl.pallas_call(
        _mm_body,
        out_shape=jax.ShapeDtypeStruct((n, mid, cols), jnp.bfloat16),
        grid=(nj, n),
        in_specs=[
            pl.BlockSpec((1, mid, hw), lambda j, i: (i, 0, 0)),
            pl.BlockSpec((hw, jt), lambda j, i: (0, j)),
        ],
        out_specs=pl.BlockSpec((1, mid, jt), lambda j, i: (i, 0, j)),
        compiler_params=_params(2, 48),
    )(xf, mf)


def _freq_body(z_ref, wrr_ref, wri_ref, wir_ref, wii_ref, br_ref, bi_ref,
               o_ref, *, mh):
    zre = z_ref[0][:, :mh]
    zim = z_ref[0][:, mh:]
    ore = (jnp.dot(wrr_ref[...], zre, preferred_element_type=jnp.float32)
           + jnp.dot(wri_ref[...], zim, preferred_element_type=jnp.float32)
           + br_ref[...].astype(jnp.float32))
    oim = (jnp.dot(wir_ref[...], zre, preferred_element_type=jnp.float32)
           + jnp.dot(wii_ref[...], zim, preferred_element_type=jnp.float32)
           + bi_ref[...].astype(jnp.float32))
    o_ref[0, :, :mh] = jnp.maximum(ore, 0.0).astype(o_ref.dtype)
    o_ref[0, :, mh:] = jnp.maximum(oim, 0.0).astype(o_ref.dtype)


def _freq_conv(y, w4, b2, *, mh):
    n, mid, cols = y.shape
    wrr, wri, wir, wii = w4
    br, bi = b2
    w_spec = pl.BlockSpec((mid, mid), lambda i: (0, 0))
    b_spec = pl.BlockSpec((mid, 1), lambda i: (0, 0))
    return pl.pallas_call(
        functools.partial(_freq_body, mh=mh),
        out_shape=jax.ShapeDtypeStruct((n, mid, cols), jnp.bfloat16),
        grid=(n,),
        in_specs=[pl.BlockSpec((1, mid, cols), lambda i: (i, 0, 0)),
                  w_spec, w_spec, w_spec, w_spec, b_spec, b_spec],
        out_specs=pl.BlockSpec((1, mid, cols), lambda i: (i, 0, 0)),
        compiler_params=_params(1, 32),
    )(y, wrr, wri, wir, wii, br, bi)


def _inv_final_body(z_ref, mi_ref, xf_ref, sp_ref, w_ref, o_ref):
    offt = jnp.dot(z_ref[0], mi_ref[...],
                   preferred_element_type=jnp.float32)
    s = (xf_ref[0].astype(jnp.float32) + offt).astype(jnp.bfloat16)
    acc = jnp.dot(w_ref[...], s, preferred_element_type=jnp.float32)
    o_ref[0] = acc + sp_ref[0].astype(jnp.float32)


def _inv_final(z, mi, xf, osp, w_out, *, jt):
    n, mid, cols = z.shape
    hw = mi.shape[1]
    c_out = w_out.shape[0]
    nj = hw // jt
    return pl.pallas_call(
        _inv_final_body,
        out_shape=jax.ShapeDtypeStruct((n, c_out, hw), jnp.float32),
        grid=(nj, n),
        in_specs=[
            pl.BlockSpec((1, mid, cols), lambda j, i: (i, 0, 0)),
            pl.BlockSpec((cols, jt), lambda j, i: (0, j)),
            pl.BlockSpec((1, mid, jt), lambda j, i: (i, 0, j)),
            pl.BlockSpec((1, c_out, jt), lambda j, i: (i, 0, j)),
            pl.BlockSpec((c_out, mid), lambda j, i: (0, 0)),
        ],
        out_specs=pl.BlockSpec((1, c_out, jt), lambda j, i: (i, 0, j)),
        compiler_params=_params(2, 48),
    )(z, mi, xf, osp, w_out)


def kernel(x, spatial_w, spatial_b, conv_in_w, conv_w, conv_b, conv_out_w):
    n, c_in, h, w = x.shape
    c_out = spatial_w.shape[0]
    mid = conv_in_w.shape[0]
    hp, wp = h + 2, w + 2
    hw = h * w
    wf = w // 2 + 1
    fpad = wf + (-(h * wf) % 128) // h
    mh = h * fpad
    lc = h * wp
    lpad = -(-(2 * wp + 2 + lc) // 128) * 128

    f32 = jnp.float32
    x_pad = jnp.pad(x, ((0, 0), (0, 0), (1, 1), (1, 1)), mode="reflect")
    x_pad_flat = x_pad.reshape(n, c_in, hp * wp).astype(jnp.bfloat16)
    x_pad_flat = jnp.pad(x_pad_flat, ((0, 0), (0, 0), (0, lpad - hp * wp)))

    w9 = (jnp.transpose(spatial_w, (2, 3, 0, 1))
          .reshape(9, c_out, c_in).astype(jnp.bfloat16))

    osp_pf, oxf_pf = _conv_in(
        x_pad_flat, w9, spatial_b.reshape(c_out, 1),
        conv_in_w.astype(jnp.bfloat16), wp=wp, lc=lc)

    osp = osp_pf.reshape(n, c_out, h, wp)[:, :, :, :w].reshape(n, c_out, hw)
    xf = oxf_pf.reshape(n, mid, h, wp)[:, :, :, :w].reshape(n, mid, hw)

    mf, mi, _, _ = _dft_mats(h, w)

    y = _dft_fwd(xf, mf, jt=mh)

    w4 = tuple(m.astype(jnp.bfloat16)
               for m in (conv_w[0::2, 0::2], conv_w[0::2, 1::2],
                         conv_w[1::2, 0::2], conv_w[1::2, 1::2]))
    b2 = (conv_b[0::2].reshape(mid, 1), conv_b[1::2].reshape(mid, 1))
    z = _freq_conv(y, w4, b2, mh=mh)

    out = _inv_final(z, mi, xf, osp, conv_out_w.astype(jnp.bfloat16),
                     jt=hw // 2)
    return out.reshape(n, c_out, h, w)


# speedup vs baseline: 2.8756x; 1.2764x over previous
import functools
import math

import jax
import jax.numpy as jnp
import numpy as np
from jax.experimental import pallas as pl
from jax.experimental.pallas import tpu as pltpu


@functools.lru_cache(maxsize=4)
def _dft_mats(h, w):
    hw = h * w
    wf = w // 2 + 1
    fpad = wf + (-(h * wf) % 128) // h
    mh = h * fpad
    inv_s = 1.0 / math.sqrt(hw)

    ang_h = (2.0 * np.pi / h) * np.outer(np.arange(h), np.arange(h))
    ang_w = (2.0 * np.pi / w) * np.outer(np.arange(w), np.arange(wf))
    ch, sh = np.cos(ang_h), np.sin(ang_h)
    cw, sw = np.cos(ang_w), np.sin(ang_w)
    zpad = ((0, 0), (0, fpad - wf))
    cwp, swp = np.pad(cw, zpad), np.pad(sw, zpad)

    m_re = (np.einsum("yk,xf->yxkf", ch, cwp)
            - np.einsum("yk,xf->yxkf", sh, swp))
    m_im = -(np.einsum("yk,xf->yxkf", sh, cwp)
             + np.einsum("yk,xf->yxkf", ch, swp))
    mf = inv_s * np.concatenate(
        [m_re.reshape(hw, mh), m_im.reshape(hw, mh)], axis=1)

    fi = np.arange(fpad)
    wt = np.where((fi == 0) | (fi == wf - 1), 1.0, 2.0)
    wt = np.where(fi < wf, wt, 0.0)
    cwi, swi = cwp * wt, swp * wt
    mi_re = (np.einsum("yk,xf->kfyx", ch, cwi)
             - np.einsum("yk,xf->kfyx", sh, swi))
    mi_im = -(np.einsum("yk,xf->kfyx", sh, cwi)
              + np.einsum("yk,xf->kfyx", ch, swi))
    mi = inv_s * np.concatenate(
        [mi_re.reshape(mh, hw), mi_im.reshape(mh, hw)], axis=0)

    bf16 = jnp.bfloat16
    return (np.asarray(mf, np.float32).astype(bf16),
            np.asarray(mi, np.float32).astype(bf16), fpad, mh)


def _params(num_axes, vmem_mb):
    return pltpu.CompilerParams(
        dimension_semantics=("parallel",) * num_axes,
        vmem_limit_bytes=vmem_mb << 20,
    )


def _conv_in_body(xp_ref, w9_ref, b_ref, win_ref, osp_ref, oxf_ref,
                  acc_ref, xfs_ref, *, h, w, wp, lc):
    ctr = None
    acc = b_ref[...].astype(jnp.float32)
    for k in range(9):
        ky, kx = divmod(k, 3)
        tap = xp_ref[0, :, ky * wp + kx:ky * wp + kx + lc]
        if k == 4:
            ctr = tap
        acc = acc + jnp.dot(w9_ref[k], tap,
                            preferred_element_type=jnp.float32)
    acc_ref[...] = acc.astype(jnp.bfloat16)
    xf = jnp.dot(win_ref[...], ctr, preferred_element_type=jnp.float32)
    xfs_ref[...] = jnp.maximum(xf, 0.0).astype(jnp.bfloat16)
    for r in range(h):
        osp_ref[0, :, r * w:(r + 1) * w] = acc_ref[:, r * wp:r * wp + w]
        oxf_ref[0, :, r * w:(r + 1) * w] = xfs_ref[:, r * wp:r * wp + w]


def _conv_in(x_pad_flat, w9, b_sp, w_in, *, h, w, wp, lc):
    n, c_in, lpad = x_pad_flat.shape
    c_out = w9.shape[1]
    mid = w_in.shape[0]
    hw = h * w
    return pl.pallas_call(
        functools.partial(_conv_in_body, h=h, w=w, wp=wp, lc=lc),
        out_shape=(jax.ShapeDtypeStruct((n, c_out, hw), jnp.bfloat16),
                   jax.ShapeDtypeStruct((n, mid, hw), jnp.bfloat16)),
        grid=(n,),
        in_specs=[
            pl.BlockSpec((1, c_in, lpad), lambda i: (i, 0, 0)),
            pl.BlockSpec((9, c_out, c_in), lambda i: (0, 0, 0)),
            pl.BlockSpec((c_out, 1), lambda i: (0, 0)),
            pl.BlockSpec((mid, c_in), lambda i: (0, 0)),
        ],
        out_specs=[
            pl.BlockSpec((1, c_out, hw), lambda i: (i, 0, 0)),
            pl.BlockSpec((1, mid, hw), lambda i: (i, 0, 0)),
        ],
        scratch_shapes=[pltpu.VMEM((c_out, lc), jnp.bfloat16),
                        pltpu.VMEM((mid, lc), jnp.bfloat16)],
        compiler_params=_params(1, 48),
    )(x_pad_flat, w9, b_sp, w_in)


def _mm_body(x_ref, m_ref, o_ref):
    o_ref[0] = jnp.dot(x_ref[0], m_ref[...],
                       preferred_element_type=jnp.float32).astype(o_ref.dtype)


def _dft_fwd(xf, mf, *, jt):
    n, mid, hw = xf.shape
    cols = mf.shape[1]
    nj = cols // jt
    return pl.pallas_call(
        _mm_body,
        out_shape=jax.ShapeDtypeStruct((n, mid, cols), jnp.bfloat16),
        grid=(nj, n),
        in_specs=[
            pl.BlockSpec((1, mid, hw), lambda j, i: (i, 0, 0)),
            pl.BlockSpec((hw, jt), lambda j, i: (0, j)),
        ],
        out_specs=pl.BlockSpec((1, mid, jt), lambda j, i: (i, 0, j)),
        compiler_params=_params(2, 48),
    )(xf, mf)


def _inv_final_body(y_ref, wrr_ref, wri_ref, wir_ref, wii_ref, br_ref, bi_ref,
                    mi_ref, xf_ref, sp_ref, w_ref, o_ref, *, mh):
    yre = y_ref[0][:, :mh]
    yim = y_ref[0][:, mh:]
    zre = jnp.maximum(
        jnp.dot(wrr_ref[...], yre, preferred_element_type=jnp.float32)
        + jnp.dot(wri_ref[...], yim, preferred_element_type=jnp.float32)
        + br_ref[...].astype(jnp.float32), 0.0).astype(jnp.bfloat16)
    zim = jnp.maximum(
        jnp.dot(wir_ref[...], yre, preferred_element_type=jnp.float32)
        + jnp.dot(wii_ref[...], yim, preferred_element_type=jnp.float32)
        + bi_ref[...].astype(jnp.float32), 0.0).astype(jnp.bfloat16)
    offt = (jnp.dot(zre, mi_ref[:mh], preferred_element_type=jnp.float32)
            + jnp.dot(zim, mi_ref[mh:], preferred_element_type=jnp.float32))
    s = (xf_ref[0].astype(jnp.float32) + offt).astype(jnp.bfloat16)
    acc = jnp.dot(w_ref[...], s, preferred_element_type=jnp.float32)
    o_ref[0] = acc + sp_ref[0].astype(jnp.float32)


def _inv_final(y, w4, b2, mi, xf, osp, w_out, *, mh, jt):
    n, mid, cols = y.shape
    hw = mi.shape[1]
    c_out = w_out.shape[0]
    nj = hw // jt
    wrr, wri, wir, wii = w4
    br, bi = b2
    w_spec = pl.BlockSpec((mid, mid), lambda j, i: (0, 0))
    b_spec = pl.BlockSpec((mid, 1), lambda j, i: (0, 0))
    return pl.pallas_call(
        functools.partial(_inv_final_body, mh=mh),
        out_shape=jax.ShapeDtypeStruct((n, c_out, hw), jnp.float32),
        grid=(nj, n),
        in_specs=[
            pl.BlockSpec((1, mid, cols), lambda j, i: (i, 0, 0)),
            w_spec, w_spec, w_spec, w_spec, b_spec, b_spec,
            pl.BlockSpec((cols, jt), lambda j, i: (0, j)),
            pl.BlockSpec((1, mid, jt), lambda j, i: (i, 0, j)),
            pl.BlockSpec((1, c_out, jt), lambda j, i: (i, 0, j)),
            pl.BlockSpec((c_out, mid), lambda j, i: (0, 0)),
        ],
        out_specs=pl.BlockSpec((1, c_out, jt), lambda j, i: (i, 0, j)),
        compiler_params=_params(2, 48),
    )(y, wrr, wri, wir, wii, br, bi, mi, xf, osp, w_out)


def kernel(x, spatial_w, spatial_b, conv_in_w, conv_w, conv_b, conv_out_w):
    n, c_in, h, w = x.shape
    c_out = spatial_w.shape[0]
    mid = conv_in_w.shape[0]
    hp, wp = h + 2, w + 2
    hw = h * w
    wf = w // 2 + 1
    fpad = wf + (-(h * wf) % 128) // h
    mh = h * fpad
    lc = h * wp
    lpad = -(-(2 * wp + 2 + lc) // 128) * 128

    f32 = jnp.float32
    x_pad = jnp.pad(x, ((0, 0), (0, 0), (1, 1), (1, 1)), mode="reflect")
    x_pad_flat = x_pad.reshape(n, c_in, hp * wp).astype(jnp.bfloat16)
    x_pad_flat = jnp.pad(x_pad_flat, ((0, 0), (0, 0), (0, lpad - hp * wp)))

    w9 = (jnp.transpose(spatial_w, (2, 3, 0, 1))
          .reshape(9, c_out, c_in).astype(jnp.bfloat16))

    osp, xf = _conv_in(
        x_pad_flat, w9, spatial_b.reshape(c_out, 1),
        conv_in_w.astype(jnp.bfloat16), h=h, w=w, wp=wp, lc=lc)

    mf, mi, _, _ = _dft_mats(h, w)

    y = _dft_fwd(xf, mf, jt=mh)

    w4 = tuple(m.astype(jnp.bfloat16)
               for m in (conv_w[0::2, 0::2], conv_w[0::2, 1::2],
                         conv_w[1::2, 0::2], conv_w[1::2, 1::2]))
    b2 = (conv_b[0::2].reshape(mid, 1), conv_b[1::2].reshape(mid, 1))

    out = _inv_final(y, w4, b2, mi, xf, osp,
                     conv_out_w.astype(jnp.bfloat16), mh=mh, jt=hw // 2)
    return out.reshape(n, c_out, h, w)


# speedup vs baseline: 2.9332x; 1.0200x over previous
import functools
import math

import jax
import jax.numpy as jnp
import numpy as np
from jax.experimental import pallas as pl
from jax.experimental.pallas import tpu as pltpu


@functools.lru_cache(maxsize=4)
def _dft_mats(h, w):
    hw = h * w
    wf = w // 2 + 1
    fpad = wf + (-(h * wf) % 128) // h
    mh = h * fpad
    inv_s = 1.0 / math.sqrt(hw)

    ang_h = (2.0 * np.pi / h) * np.outer(np.arange(h), np.arange(h))
    ang_w = (2.0 * np.pi / w) * np.outer(np.arange(w), np.arange(wf))
    ch, sh = np.cos(ang_h), np.sin(ang_h)
    cw, sw = np.cos(ang_w), np.sin(ang_w)
    zpad = ((0, 0), (0, fpad - wf))
    cwp, swp = np.pad(cw, zpad), np.pad(sw, zpad)

    m_re = (np.einsum("yk,xf->yxkf", ch, cwp)
            - np.einsum("yk,xf->yxkf", sh, swp))
    m_im = -(np.einsum("yk,xf->yxkf", sh, cwp)
             + np.einsum("yk,xf->yxkf", ch, swp))
    mf = inv_s * np.concatenate(
        [m_re.reshape(hw, mh), m_im.reshape(hw, mh)], axis=1)

    fi = np.arange(fpad)
    wt = np.where((fi == 0) | (fi == wf - 1), 1.0, 2.0)
    wt = np.where(fi < wf, wt, 0.0)
    cwi, swi = cwp * wt, swp * wt
    mi_re = (np.einsum("yk,xf->kfyx", ch, cwi)
             - np.einsum("yk,xf->kfyx", sh, swi))
    mi_im = -(np.einsum("yk,xf->kfyx", sh, cwi)
              + np.einsum("yk,xf->kfyx", ch, swi))
    mi = inv_s * np.concatenate(
        [mi_re.reshape(mh, hw), mi_im.reshape(mh, hw)], axis=0)

    bf16 = jnp.bfloat16
    return (np.asarray(mf, np.float32).astype(bf16),
            np.asarray(mi, np.float32).astype(bf16), fpad, mh)


def _params(num_axes, vmem_mb):
    return pltpu.CompilerParams(
        dimension_semantics=("parallel",) * num_axes,
        vmem_limit_bytes=vmem_mb << 20,
    )


def _conv_in_body(x_ref, w9_ref, b_ref, win_ref, osp_ref, oxf_ref,
                  xp_ref, acc_ref, xfs_ref, *, h, w, wp, lc, lpad):
    hp = h + 2
    xb = x_ref[0].astype(jnp.bfloat16)
    for r in range(h):
        base = (r + 1) * wp
        rw = r * w
        xp_ref[:, base + 1:base + 1 + w] = xb[:, rw:rw + w]
        xp_ref[:, base:base + 1] = xb[:, rw + 1:rw + 2]
        xp_ref[:, base + w + 1:base + w + 2] = xb[:, rw + w - 2:rw + w - 1]
    xp_ref[:, 0:wp] = xp_ref[:, 2 * wp:3 * wp]
    xp_ref[:, (hp - 1) * wp:hp * wp] = xp_ref[:, (hp - 3) * wp:(hp - 2) * wp]
    xp_ref[:, hp * wp:] = jnp.zeros_like(xp_ref[:, hp * wp:])

    ctr = None
    acc = b_ref[...].astype(jnp.float32)
    for k in range(9):
        ky, kx = divmod(k, 3)
        tap = xp_ref[:, ky * wp + kx:ky * wp + kx + lc]
        if k == 4:
            ctr = tap
        acc = acc + jnp.dot(w9_ref[k], tap,
                            preferred_element_type=jnp.float32)
    acc_ref[...] = acc.astype(jnp.bfloat16)
    xf = jnp.dot(win_ref[...], ctr, preferred_element_type=jnp.float32)
    xfs_ref[...] = jnp.maximum(xf, 0.0).astype(jnp.bfloat16)
    for r in range(h):
        osp_ref[0, :, r * w:(r + 1) * w] = acc_ref[:, r * wp:r * wp + w]
        oxf_ref[0, :, r * w:(r + 1) * w] = xfs_ref[:, r * wp:r * wp + w]


def _conv_in(x, w9, b_sp, w_in, *, h, w, wp, lc, lpad):
    n, c_in, hw = x.shape
    c_out = w9.shape[1]
    mid = w_in.shape[0]
    return pl.pallas_call(
        functools.partial(_conv_in_body, h=h, w=w, wp=wp, lc=lc, lpad=lpad),
        out_shape=(jax.ShapeDtypeStruct((n, c_out, hw), jnp.bfloat16),
                   jax.ShapeDtypeStruct((n, mid, hw), jnp.bfloat16)),
        grid=(n,),
        in_specs=[
            pl.BlockSpec((1, c_in, hw), lambda i: (i, 0, 0)),
            pl.BlockSpec((9, c_out, c_in), lambda i: (0, 0, 0)),
            pl.BlockSpec((c_out, 1), lambda i: (0, 0)),
            pl.BlockSpec((mid, c_in), lambda i: (0, 0)),
        ],
        out_specs=[
            pl.BlockSpec((1, c_out, hw), lambda i: (i, 0, 0)),
            pl.BlockSpec((1, mid, hw), lambda i: (i, 0, 0)),
        ],
        scratch_shapes=[pltpu.VMEM((c_in, lpad), jnp.bfloat16),
                        pltpu.VMEM((c_out, lc), jnp.bfloat16),
                        pltpu.VMEM((mid, lc), jnp.bfloat16)],
        compiler_params=_params(1, 48),
    )(x, w9, b_sp, w_in)


def _mm_body(x_ref, m_ref, o_ref):
    o_ref[0] = jnp.dot(x_ref[0], m_ref[...],
                       preferred_element_type=jnp.float32).astype(o_ref.dtype)


def _dft_fwd(xf, mf, *, jt):
    n, mid, hw = xf.shape
    cols = mf.shape[1]
    nj = cols // jt
    return pl.pallas_call(
        _mm_body,
        out_shape=jax.ShapeDtypeStruct((n, mid, cols), jnp.bfloat16),
        grid=(nj, n),
        in_specs=[
            pl.BlockSpec((1, mid, hw), lambda j, i: (i, 0, 0)),
            pl.BlockSpec((hw, jt), lambda j, i: (0, j)),
        ],
        out_specs=pl.BlockSpec((1, mid, jt), lambda j, i: (i, 0, j)),
        compiler_params=_params(2, 48),
    )(xf, mf)


def _inv_final_body(y_ref, wrr_ref, wri_ref, wir_ref, wii_ref, br_ref, bi_ref,
                    mi_ref, xf_ref, sp_ref, w_ref, o_ref, *, mh):
    yre = y_ref[0][:, :mh]
    yim = y_ref[0][:, mh:]
    zre = jnp.maximum(
        jnp.dot(wrr_ref[...], yre, preferred_element_type=jnp.float32)
        + jnp.dot(wri_ref[...], yim, preferred_element_type=jnp.float32)
        + br_ref[...].astype(jnp.float32), 0.0).astype(jnp.bfloat16)
    zim = jnp.maximum(
        jnp.dot(wir_ref[...], yre, preferred_element_type=jnp.float32)
        + jnp.dot(wii_ref[...], yim, preferred_element_type=jnp.float32)
        + bi_ref[...].astype(jnp.float32), 0.0).astype(jnp.bfloat16)
    offt = (jnp.dot(zre, mi_ref[:mh], preferred_element_type=jnp.float32)
            + jnp.dot(zim, mi_ref[mh:], preferred_element_type=jnp.float32))
    s = (xf_ref[0].astype(jnp.float32) + offt).astype(jnp.bfloat16)
    acc = jnp.dot(w_ref[...], s, preferred_element_type=jnp.float32)
    o_ref[0] = acc + sp_ref[0].astype(jnp.float32)


def _inv_final(y, w4, b2, mi, xf, osp, w_out, *, mh, jt):
    n, mid, cols = y.shape
    hw = mi.shape[1]
    c_out = w_out.shape[0]
    nj = hw // jt
    wrr, wri, wir, wii = w4
    br, bi = b2
    w_spec = pl.BlockSpec((mid, mid), lambda j, i: (0, 0))
    b_spec = pl.BlockSpec((mid, 1), lambda j, i: (0, 0))
    return pl.pallas_call(
        functools.partial(_inv_final_body, mh=mh),
        out_shape=jax.ShapeDtypeStruct((n, c_out, hw), jnp.float32),
        grid=(nj, n),
        in_specs=[
            pl.BlockSpec((1, mid, cols), lambda j, i: (i, 0, 0)),
            w_spec, w_spec, w_spec, w_spec, b_spec, b_spec,
            pl.BlockSpec((cols, jt), lambda j, i: (0, j)),
            pl.BlockSpec((1, mid, jt), lambda j, i: (i, 0, j)),
            pl.BlockSpec((1, c_out, jt), lambda j, i: (i, 0, j)),
            pl.BlockSpec((c_out, mid), lambda j, i: (0, 0)),
        ],
        out_specs=pl.BlockSpec((1, c_out, jt), lambda j, i: (i, 0, j)),
        compiler_params=_params(2, 48),
    )(y, wrr, wri, wir, wii, br, bi, mi, xf, osp, w_out)


def kernel(x, spatial_w, spatial_b, conv_in_w, conv_w, conv_b, conv_out_w):
    n, c_in, h, w = x.shape
    c_out = spatial_w.shape[0]
    mid = conv_in_w.shape[0]
    hp, wp = h + 2, w + 2
    hw = h * w
    wf = w // 2 + 1
    fpad = wf + (-(h * wf) % 128) // h
    mh = h * fpad
    lc = h * wp
    lpad = -(-(2 * wp + 2 + lc) // 128) * 128

    w9 = (jnp.transpose(spatial_w, (2, 3, 0, 1))
          .reshape(9, c_out, c_in).astype(jnp.bfloat16))

    osp, xf = _conv_in(
        x.reshape(n, c_in, hw), w9, spatial_b.reshape(c_out, 1),
        conv_in_w.astype(jnp.bfloat16), h=h, w=w, wp=wp, lc=lc, lpad=lpad)

    mf, mi, _, _ = _dft_mats(h, w)

    y = _dft_fwd(xf, mf, jt=mh)

    w4 = tuple(m.astype(jnp.bfloat16)
               for m in (conv_w[0::2, 0::2], conv_w[0::2, 1::2],
                         conv_w[1::2, 0::2], conv_w[1::2, 1::2]))
    b2 = (conv_b[0::2].reshape(mid, 1), conv_b[1::2].reshape(mid, 1))

    out = _inv_final(y, w4, b2, mi, xf, osp,
                     conv_out_w.astype(jnp.bfloat16), mh=mh, jt=hw // 2)
    return out.reshape(n, c_out, h, w)
